# baseline (speedup 1.0000x reference)
import functools

import jax
import jax.numpy as jnp
import numpy as np
from jax import lax
from jax.experimental import pallas as pl
from jax.experimental.pallas import tpu as pltpu
from jax.experimental.pallas import tpu_sc as plsc

NC, NS, L = 2, 16, 16
NW = NC * NS
PH, PW = 7, 7
CELLS = PH * PW
IDXW = 104

HS = (256, 128, 64, 32)
BASES = (0, 65536, 81920, 86016)

FRAC = tuple(float(np.float32(k) / np.float32(6.0)) for k in range(PH))


@functools.lru_cache(maxsize=None)
def _build_sc_call(n, n_pad, c):
    r_per_w = n_pad // NW
    g_per_w = r_per_w // L
    cw = c // 2
    cj = cw // L

    mesh = plsc.VectorSubcoreMesh(core_axis_name="c", subcore_axis_name="s",
                                  num_cores=NC, num_subcores=NS)

    def body(table, rois_t, pads, out, out_dummy, roi_v, pads_v, idx_buf,
             wy_buf, wx_buf, ab0, cd0, ab1, cd1, out_buf, sem0, sem1, osem):
        cid = lax.axis_index("c")
        sid = lax.axis_index("s")
        wid = sid * NC + cid
        base_roi = wid * r_per_w

        for i in range(4):
            pltpu.sync_copy(rois_t.at[pl.ds(i * n_pad + base_roi, r_per_w)],
                            roi_v.at[pl.ds(i * r_per_w, r_per_w)])
        pltpu.sync_copy(pads, pads_v)
        pads_vec = pads_v[...]
        area = pads_vec[0] * pads_vec[1]
        t3 = 224.0 * 224.0 * 0.125
        t4 = 224.0 * 224.0 * 0.5
        t5 = 224.0 * 224.0 * 2.0

        zero16 = jnp.zeros((L,), jnp.int32)
        for r in range(L * 2):
            for off in (0, 16, 32, 48, 64, 80, 88):
                idx_buf[r, pl.ds(off, L)] = zero16

        lanes = lax.iota(jnp.int32, L)
        rows_ab = lanes * 2
        one = jnp.ones((L,), jnp.int32)
        sets = ((ab0, cd0, sem0), (ab1, cd1, sem1))

        def fire(rr, s):
            abuf, cbuf, sem = sets[s]
            pltpu.async_copy(table.at[idx_buf.at[rr * 2]], abuf, sem)
            pltpu.async_copy(table.at[idx_buf.at[rr * 2 + 1]], cbuf, sem)

        def wait(s):
            abuf, cbuf, sem = sets[s]
            pltpu.make_async_copy(table.at[idx_buf.at[0]], abuf, sem).wait()
            pltpu.make_async_copy(table.at[idx_buf.at[1]], cbuf, sem).wait()

        def wait_out():
            pltpu.make_async_copy(out_buf, out.at[0], osem).wait()

        def combine(rr, s, goff):
            abuf, cbuf, _ = sets[s]
            rs = jnp.full((L,), rr, jnp.int32)
            roi_g = base_roi + goff + rr

            @pl.when(goff + rr > 0)
            def _():
                wait_out()

            def cell_body(cc, carry3):
                pyv = cc // PW
                pxv = cc - pyv * PW
                wys = plsc.load_gather(
                    wy_buf, [jnp.full((L,), pyv, jnp.int32), rs])
                wxs = plsc.load_gather(
                    wx_buf, [jnp.full((L,), pxv, jnp.int32), rs])
                ilv = plsc.PackFormat.INTERLEAVED
                for j in range(cj):
                    sl = pl.ds(j * L, L)
                    ae, ao = plsc.unpack(
                        plsc.bitcast(abuf[cc, sl], jnp.bfloat16), format=ilv)
                    be, bo = plsc.unpack(
                        plsc.bitcast(abuf[CELLS + cc, sl], jnp.bfloat16),
                        format=ilv)
                    ce, co = plsc.unpack(
                        plsc.bitcast(cbuf[cc, sl], jnp.bfloat16), format=ilv)
                    de, do_ = plsc.unpack(
                        plsc.bitcast(cbuf[CELLS + cc, sl], jnp.bfloat16),
                        format=ilv)
                    tope = ae + wxs * (be - ae)
                    bote = ce + wxs * (de - ce)
                    oute = tope + wys * (bote - tope)
                    topo = ao + wxs * (bo - ao)
                    boto = co + wxs * (do_ - co)
                    outo = topo + wys * (boto - topo)
                    out_buf[cc, sl] = plsc.bitcast(
                        plsc.pack(oute, outo, format=ilv), jnp.int32)
                return carry3

            lax.fori_loop(0, CELLS, cell_body, 0)

            @pl.when(roi_g < n)
            def _():
                pltpu.async_copy(out_buf, out.at[roi_g], osem)

            @pl.when(roi_g >= n)
            def _():
                pltpu.async_copy(out_buf, out_dummy, osem)

        def group_body(g, carry):
            goff = g * L
            y1v = roi_v[pl.ds(goff, L)]
            x1v = roi_v[pl.ds(r_per_w + goff, L)]
            y2v = roi_v[pl.ds(2 * r_per_w + goff, L)]
            x2v = roi_v[pl.ds(3 * r_per_w + goff, L)]
            hv = y2v - y1v
            wv = x2v - x1v
            sv = hv * wv * area
            li = (jnp.where(sv >= t3, one, 0) + jnp.where(sv >= t4, one, 0)
                  + jnp.where(sv >= t5, one, 0))
            hm1f = jnp.where(li == 0, float(HS[0] - 1),
                             jnp.where(li == 1, float(HS[1] - 1),
                                       jnp.where(li == 2, float(HS[2] - 1),
                                                 float(HS[3] - 1))))
            hm1i = jnp.where(li == 0, HS[0] - 1,
                             jnp.where(li == 1, HS[1] - 1,
                                       jnp.where(li == 2, HS[2] - 1,
                                                 HS[3] - 1)))
            wint = hm1i + 1
            basev = jnp.where(li == 0, BASES[0],
                              jnp.where(li == 1, BASES[1],
                                        jnp.where(li == 2, BASES[2],
                                                  BASES[3])))
            hh = hv * hm1f
            ww = wv * hm1f
            y1s = y1v * hm1f
            x1s = x1v * hm1f

            ya = []
            yb = []
            x0 = []
            x1 = []
            for k in range(PH):
                ys = y1s + FRAC[k] * hh
                y0i = ys.astype(jnp.int32)
                wy_buf[k] = ys - y0i.astype(jnp.float32)
                y1i = jnp.minimum(y0i + 1, hm1i)
                ya.append(basev + y0i * wint)
                yb.append(basev + y1i * wint)
                xs = x1s + FRAC[k] * ww
                x0i = xs.astype(jnp.int32)
                wx_buf[k] = xs - x0i.astype(jnp.float32)
                x1.append(jnp.minimum(x0i + 1, hm1i))
                x0.append(x0i)

            for py in range(PH):
                for px in range(PW):
                    cc = py * PW + px
                    col_a = jnp.full((L,), cc, jnp.int32)
                    col_b = jnp.full((L,), CELLS + cc, jnp.int32)
                    plsc.store_scatter(idx_buf, [rows_ab, col_a],
                                       ya[py] + x0[px])
                    plsc.store_scatter(idx_buf, [rows_ab, col_b],
                                       ya[py] + x1[px])
                    plsc.store_scatter(idx_buf, [rows_ab + one, col_a],
                                       yb[py] + x0[px])
                    plsc.store_scatter(idx_buf, [rows_ab + one, col_b],
                                       yb[py] + x1[px])

            fire(0, 0)

            def roi_pair(rh, carry2):
                r0 = rh * 2
                wait(0)
                fire(r0 + 1, 1)
                combine(r0, 0, goff)
                wait(1)

                @pl.when(r0 < L - 2)
                def _():
                    fire(r0 + 2, 0)

                combine(r0 + 1, 1, goff)
                return carry2

            lax.fori_loop(0, L // 2, roi_pair, 0)
            return carry

        lax.fori_loop(0, g_per_w, group_body, 0)
        wait_out()

    return pl.kernel(
        body,
        out_type=(jax.ShapeDtypeStruct((n, CELLS, cw), jnp.int32),
                  jax.ShapeDtypeStruct((CELLS, cw), jnp.int32)),
        mesh=mesh,
        scratch_types=[
            pltpu.VMEM((4 * r_per_w,), jnp.float32),
            pltpu.VMEM((L,), jnp.float32),
            pltpu.VMEM((L * 2, IDXW), jnp.int32),
            pltpu.VMEM((PH, L), jnp.float32),
            pltpu.VMEM((PW, L), jnp.float32),
            pltpu.VMEM((IDXW, cw), jnp.int32),
            pltpu.VMEM((IDXW, cw), jnp.int32),
            pltpu.VMEM((IDXW, cw), jnp.int32),
            pltpu.VMEM((IDXW, cw), jnp.int32),
            pltpu.VMEM((CELLS, cw), jnp.int32),
            pltpu.SemaphoreType.DMA,
            pltpu.SemaphoreType.DMA,
            pltpu.SemaphoreType.DMA,
        ],
        compiler_params=pltpu.CompilerParams(needs_layout_passes=False),
    )


def kernel(rois, feat_p2, feat_p3, feat_p4, feat_p5, img_metas):
    n = rois.shape[0]
    c = feat_p2.shape[-1]
    n_pad = ((n + NW * L - 1) // (NW * L)) * (NW * L)

    table = lax.bitcast_convert_type(
        jnp.concatenate(
            [f.reshape(-1, c) for f in (feat_p2, feat_p3, feat_p4, feat_p5)],
            axis=0).astype(jnp.bfloat16).reshape(-1, c // 2, 2),
        jnp.int32)
    rois_t = jnp.zeros((4, n_pad), jnp.float32).at[:, :n].set(
        rois.astype(jnp.float32).T).reshape(-1)
    pads = jnp.zeros((L,), jnp.float32).at[0].set(
        img_metas[0, 7]).at[1].set(img_metas[0, 8])

    out, _ = _build_sc_call(n, n_pad, c)(table, rois_t, pads)
    return lax.bitcast_convert_type(out, jnp.bfloat16).astype(
        jnp.float32).reshape(n, PH, PW, c)


# speedup vs baseline: 1.0235x; 1.0235x over previous
import functools

import jax
import jax.numpy as jnp
import numpy as np
from jax import lax
from jax.experimental import pallas as pl
from jax.experimental.pallas import tpu as pltpu
from jax.experimental.pallas import tpu_sc as plsc

NC, NS, L = 2, 16, 16
NW = NC * NS
PH, PW = 7, 7
CELLS = PH * PW
IDXW = 104

HS = (256, 128, 64, 32)
BASES = (0, 65536, 81920, 86016)

FRAC = tuple(float(np.float32(k) / np.float32(6.0)) for k in range(PH))


@functools.lru_cache(maxsize=None)
def _build_sc_call(n, n_pad, c):
    r_per_w = n_pad // NW
    g_per_w = r_per_w // L
    cw = c // 2
    cj = cw // L

    mesh = plsc.VectorSubcoreMesh(core_axis_name="c", subcore_axis_name="s",
                                  num_cores=NC, num_subcores=NS)

    def body(table, rois_t, pads, out, out_dummy, roi_v, pads_v, idx_buf,
             wy_buf, wx_buf, ab0, cd0, ab1, cd1, out_buf, sem0, sem1, osem):
        cid = lax.axis_index("c")
        sid = lax.axis_index("s")
        wid = sid * NC + cid
        base_roi = wid * r_per_w

        for i in range(4):
            pltpu.sync_copy(rois_t.at[pl.ds(i * n_pad + base_roi, r_per_w)],
                            roi_v.at[pl.ds(i * r_per_w, r_per_w)])
        pltpu.sync_copy(pads, pads_v)
        pads_vec = pads_v[...]
        area = pads_vec[0] * pads_vec[1]
        t3 = 224.0 * 224.0 * 0.125
        t4 = 224.0 * 224.0 * 0.5
        t5 = 224.0 * 224.0 * 2.0

        zero16 = jnp.zeros((L,), jnp.int32)
        for r in range(L * 2):
            for off in (0, 16, 32, 48, 64, 80, 88):
                idx_buf[r, pl.ds(off, L)] = zero16

        lanes = lax.iota(jnp.int32, L)
        rows_ab = lanes * 2
        one = jnp.ones((L,), jnp.int32)
        sets = ((ab0, cd0, sem0), (ab1, cd1, sem1))

        def fire(rr, s):
            abuf, cbuf, sem = sets[s]
            pltpu.async_copy(table.at[idx_buf.at[rr * 2]], abuf, sem)
            pltpu.async_copy(table.at[idx_buf.at[rr * 2 + 1]], cbuf, sem)

        def wait(s):
            abuf, cbuf, sem = sets[s]
            pltpu.make_async_copy(table.at[idx_buf.at[0]], abuf, sem).wait()
            pltpu.make_async_copy(table.at[idx_buf.at[1]], cbuf, sem).wait()

        def wait_out():
            pltpu.make_async_copy(out_buf, out.at[0], osem).wait()

        def combine(rr, s, goff):
            abuf, cbuf, _ = sets[s]
            rs = jnp.full((L,), rr, jnp.int32)
            roi_g = base_roi + goff + rr

            @pl.when(goff + rr > 0)
            def _():
                wait_out()

            def cell_body(cc, carry3):
                pyv = cc // PW
                pxv = cc - pyv * PW
                wys = plsc.load_gather(
                    wy_buf, [jnp.full((L,), pyv, jnp.int32), rs])
                wxs = plsc.load_gather(
                    wx_buf, [jnp.full((L,), pxv, jnp.int32), rs])
                wd = wys * wxs
                wc = wys - wd
                wb = wxs - wd
                wa = (1.0 - wxs) - wc
                ilv = plsc.PackFormat.INTERLEAVED
                wab = plsc.pack(wa, wa, format=ilv)
                wbb = plsc.pack(wb, wb, format=ilv)
                wcb = plsc.pack(wc, wc, format=ilv)
                wdb = plsc.pack(wd, wd, format=ilv)
                for j in range(cj):
                    sl = pl.ds(j * L, L)
                    av = plsc.bitcast(abuf[cc, sl], jnp.bfloat16)
                    bv = plsc.bitcast(abuf[CELLS + cc, sl], jnp.bfloat16)
                    cv = plsc.bitcast(cbuf[cc, sl], jnp.bfloat16)
                    dv = plsc.bitcast(cbuf[CELLS + cc, sl], jnp.bfloat16)
                    res = av * wab + bv * wbb + cv * wcb + dv * wdb
                    out_buf[cc, sl] = plsc.bitcast(res, jnp.int32)
                return carry3

            lax.fori_loop(0, CELLS, cell_body, 0)

            @pl.when(roi_g < n)
            def _():
                pltpu.async_copy(out_buf, out.at[roi_g], osem)

            @pl.when(roi_g >= n)
            def _():
                pltpu.async_copy(out_buf, out_dummy, osem)

        def group_body(g, carry):
            goff = g * L
            y1v = roi_v[pl.ds(goff, L)]
            x1v = roi_v[pl.ds(r_per_w + goff, L)]
            y2v = roi_v[pl.ds(2 * r_per_w + goff, L)]
            x2v = roi_v[pl.ds(3 * r_per_w + goff, L)]
            hv = y2v - y1v
            wv = x2v - x1v
            sv = hv * wv * area
            li = (jnp.where(sv >= t3, one, 0) + jnp.where(sv >= t4, one, 0)
                  + jnp.where(sv >= t5, one, 0))
            hm1f = jnp.where(li == 0, float(HS[0] - 1),
                             jnp.where(li == 1, float(HS[1] - 1),
                                       jnp.where(li == 2, float(HS[2] - 1),
                                                 float(HS[3] - 1))))
            hm1i = jnp.where(li == 0, HS[0] - 1,
                             jnp.where(li == 1, HS[1] - 1,
                                       jnp.where(li == 2, HS[2] - 1,
                                                 HS[3] - 1)))
            wint = hm1i + 1
            basev = jnp.where(li == 0, BASES[0],
                              jnp.where(li == 1, BASES[1],
                                        jnp.where(li == 2, BASES[2],
                                                  BASES[3])))
            hh = hv * hm1f
            ww = wv * hm1f
            y1s = y1v * hm1f
            x1s = x1v * hm1f

            ya = []
            yb = []
            x0 = []
            x1 = []
            for k in range(PH):
                ys = y1s + FRAC[k] * hh
                y0i = ys.astype(jnp.int32)
                wy_buf[k] = ys - y0i.astype(jnp.float32)
                y1i = jnp.minimum(y0i + 1, hm1i)
                ya.append(basev + y0i * wint)
                yb.append(basev + y1i * wint)
                xs = x1s + FRAC[k] * ww
                x0i = xs.astype(jnp.int32)
                wx_buf[k] = xs - x0i.astype(jnp.float32)
                x1.append(jnp.minimum(x0i + 1, hm1i))
                x0.append(x0i)

            for py in range(PH):
                for px in range(PW):
                    cc = py * PW + px
                    col_a = jnp.full((L,), cc, jnp.int32)
                    col_b = jnp.full((L,), CELLS + cc, jnp.int32)
                    plsc.store_scatter(idx_buf, [rows_ab, col_a],
                                       ya[py] + x0[px])
                    plsc.store_scatter(idx_buf, [rows_ab, col_b],
                                       ya[py] + x1[px])
                    plsc.store_scatter(idx_buf, [rows_ab + one, col_a],
                                       yb[py] + x0[px])
                    plsc.store_scatter(idx_buf, [rows_ab + one, col_b],
                                       yb[py] + x1[px])

            fire(0, 0)

            def roi_pair(rh, carry2):
                r0 = rh * 2
                wait(0)
                fire(r0 + 1, 1)
                combine(r0, 0, goff)
                wait(1)

                @pl.when(r0 < L - 2)
                def _():
                    fire(r0 + 2, 0)

                combine(r0 + 1, 1, goff)
                return carry2

            lax.fori_loop(0, L // 2, roi_pair, 0)
            return carry

        lax.fori_loop(0, g_per_w, group_body, 0)
        wait_out()

    return pl.kernel(
        body,
        out_type=(jax.ShapeDtypeStruct((n, CELLS, cw), jnp.int32),
                  jax.ShapeDtypeStruct((CELLS, cw), jnp.int32)),
        mesh=mesh,
        scratch_types=[
            pltpu.VMEM((4 * r_per_w,), jnp.float32),
            pltpu.VMEM((L,), jnp.float32),
            pltpu.VMEM((L * 2, IDXW), jnp.int32),
            pltpu.VMEM((PH, L), jnp.float32),
            pltpu.VMEM((PW, L), jnp.float32),
            pltpu.VMEM((IDXW, cw), jnp.int32),
            pltpu.VMEM((IDXW, cw), jnp.int32),
            pltpu.VMEM((IDXW, cw), jnp.int32),
            pltpu.VMEM((IDXW, cw), jnp.int32),
            pltpu.VMEM((CELLS, cw), jnp.int32),
            pltpu.SemaphoreType.DMA,
            pltpu.SemaphoreType.DMA,
            pltpu.SemaphoreType.DMA,
        ],
        compiler_params=pltpu.CompilerParams(needs_layout_passes=False),
    )


def kernel(rois, feat_p2, feat_p3, feat_p4, feat_p5, img_metas):
    n = rois.shape[0]
    c = feat_p2.shape[-1]
    n_pad = ((n + NW * L - 1) // (NW * L)) * (NW * L)

    table = lax.bitcast_convert_type(
        jnp.concatenate(
            [f.reshape(-1, c) for f in (feat_p2, feat_p3, feat_p4, feat_p5)],
            axis=0).astype(jnp.bfloat16).reshape(-1, c // 2, 2),
        jnp.int32)
    rois_t = jnp.zeros((4, n_pad), jnp.float32).at[:, :n].set(
        rois.astype(jnp.float32).T).reshape(-1)
    pads = jnp.zeros((L,), jnp.float32).at[0].set(
        img_metas[0, 7]).at[1].set(img_metas[0, 8])

    out, _ = _build_sc_call(n, n_pad, c)(table, rois_t, pads)
    return lax.bitcast_convert_type(out, jnp.bfloat16).astype(
        jnp.float32).reshape(n, PH, PW, c)


# speedup vs baseline: 1.2802x; 1.2508x over previous
import functools

import jax
import jax.numpy as jnp
import numpy as np
from jax import lax
from jax.experimental import pallas as pl
from jax.experimental.pallas import tpu as pltpu
from jax.experimental.pallas import tpu_sc as plsc

NC, NS, L = 2, 16, 16
NW = NC * NS
PH, PW = 7, 7
CELLS = PH * PW
IDXW = 104

HS = (256, 128, 64, 32)
BASES = (0, 65536, 81920, 86016)

FRAC = tuple(float(np.float32(k) / np.float32(6.0)) for k in range(PH))


@functools.lru_cache(maxsize=None)
def _build_sc_call(n, n_pad, c):
    r_per_w = n_pad // NW
    g_per_w = r_per_w // L
    cj = c // L

    mesh = plsc.VectorSubcoreMesh(core_axis_name="c", subcore_axis_name="s",
                                  num_cores=NC, num_subcores=NS)

    def body(table, rois_t, pads, out, out_dummy, roi_v, pads_v, idx_buf,
             wy_buf, wx_buf, ab0, cd0, ab1, cd1, out_buf, sem0, sem1, osem):
        cid = lax.axis_index("c")
        sid = lax.axis_index("s")
        wid = sid * NC + cid
        base_roi = wid * r_per_w

        for i in range(4):
            pltpu.sync_copy(rois_t.at[pl.ds(i * n_pad + base_roi, r_per_w)],
                            roi_v.at[pl.ds(i * r_per_w, r_per_w)])
        pltpu.sync_copy(pads, pads_v)
        pads_vec = pads_v[...]
        area = pads_vec[0] * pads_vec[1]
        t3 = 224.0 * 224.0 * 0.125
        t4 = 224.0 * 224.0 * 0.5
        t5 = 224.0 * 224.0 * 2.0

        zero16 = jnp.zeros((L,), jnp.int32)
        for r in range(L * 2):
            for off in (0, 16, 32, 48, 64, 80, 88):
                idx_buf[r, pl.ds(off, L)] = zero16

        lanes = lax.iota(jnp.int32, L)
        rows_ab = lanes * 2
        one = jnp.ones((L,), jnp.int32)
        sets = ((ab0, cd0, sem0), (ab1, cd1, sem1))

        def fire(rr, s):
            abuf, cbuf, sem = sets[s]
            pltpu.async_copy(table.at[idx_buf.at[rr * 2]], abuf, sem)
            pltpu.async_copy(table.at[idx_buf.at[rr * 2 + 1]], cbuf, sem)

        def wait(s):
            abuf, cbuf, sem = sets[s]
            pltpu.make_async_copy(table.at[idx_buf.at[0]], abuf, sem).wait()
            pltpu.make_async_copy(table.at[idx_buf.at[1]], cbuf, sem).wait()

        def wait_out():
            pltpu.make_async_copy(out_buf, out.at[0], osem).wait()

        def combine(rr, s, goff):
            abuf, cbuf, _ = sets[s]
            rs = jnp.full((L,), rr, jnp.int32)
            roi_g = base_roi + goff + rr

            @pl.when(goff + rr > 0)
            def _():
                wait_out()

            def cell_body(cc, carry3):
                pyv = cc // PW
                pxv = cc - pyv * PW
                wys = plsc.load_gather(
                    wy_buf, [jnp.full((L,), pyv, jnp.int32), rs])
                wxs = plsc.load_gather(
                    wx_buf, [jnp.full((L,), pxv, jnp.int32), rs])
                wd = wys * wxs
                wc = wys - wd
                wb = wxs - wd
                wa = (1.0 - wxs) - wc
                for j in range(cj):
                    sl = pl.ds(j * L, L)
                    av = abuf[cc, sl]
                    bv = abuf[CELLS + cc, sl]
                    cv = cbuf[cc, sl]
                    dv = cbuf[CELLS + cc, sl]
                    out_buf[cc, sl] = (av * wa + bv * wb + cv * wc + dv * wd)
                return carry3

            lax.fori_loop(0, CELLS, cell_body, 0)

            @pl.when(roi_g < n)
            def _():
                pltpu.async_copy(out_buf, out.at[roi_g], osem)

            @pl.when(roi_g >= n)
            def _():
                pltpu.async_copy(out_buf, out_dummy, osem)

        def group_body(g, carry):
            goff = g * L
            y1v = roi_v[pl.ds(goff, L)]
            x1v = roi_v[pl.ds(r_per_w + goff, L)]
            y2v = roi_v[pl.ds(2 * r_per_w + goff, L)]
            x2v = roi_v[pl.ds(3 * r_per_w + goff, L)]
            hv = y2v - y1v
            wv = x2v - x1v
            sv = hv * wv * area
            li = (jnp.where(sv >= t3, one, 0) + jnp.where(sv >= t4, one, 0)
                  + jnp.where(sv >= t5, one, 0))
            hm1f = jnp.where(li == 0, float(HS[0] - 1),
                             jnp.where(li == 1, float(HS[1] - 1),
                                       jnp.where(li == 2, float(HS[2] - 1),
                                                 float(HS[3] - 1))))
            hm1i = jnp.where(li == 0, HS[0] - 1,
                             jnp.where(li == 1, HS[1] - 1,
                                       jnp.where(li == 2, HS[2] - 1,
                                                 HS[3] - 1)))
            wint = hm1i + 1
            basev = jnp.where(li == 0, BASES[0],
                              jnp.where(li == 1, BASES[1],
                                        jnp.where(li == 2, BASES[2],
                                                  BASES[3])))
            hh = hv * hm1f
            ww = wv * hm1f
            y1s = y1v * hm1f
            x1s = x1v * hm1f

            ya = []
            yb = []
            x0 = []
            x1 = []
            for k in range(PH):
                ys = y1s + FRAC[k] * hh
                y0i = ys.astype(jnp.int32)
                wy_buf[k] = ys - y0i.astype(jnp.float32)
                y1i = jnp.minimum(y0i + 1, hm1i)
                ya.append(basev + y0i * wint)
                yb.append(basev + y1i * wint)
                xs = x1s + FRAC[k] * ww
                x0i = xs.astype(jnp.int32)
                wx_buf[k] = xs - x0i.astype(jnp.float32)
                x1.append(jnp.minimum(x0i + 1, hm1i))
                x0.append(x0i)

            for py in range(PH):
                for px in range(PW):
                    cc = py * PW + px
                    col_a = jnp.full((L,), cc, jnp.int32)
                    col_b = jnp.full((L,), CELLS + cc, jnp.int32)
                    plsc.store_scatter(idx_buf, [rows_ab, col_a],
                                       ya[py] + x0[px])
                    plsc.store_scatter(idx_buf, [rows_ab, col_b],
                                       ya[py] + x1[px])
                    plsc.store_scatter(idx_buf, [rows_ab + one, col_a],
                                       yb[py] + x0[px])
                    plsc.store_scatter(idx_buf, [rows_ab + one, col_b],
                                       yb[py] + x1[px])

            fire(0, 0)

            def roi_pair(rh, carry2):
                r0 = rh * 2
                wait(0)
                fire(r0 + 1, 1)
                combine(r0, 0, goff)
                wait(1)

                @pl.when(r0 < L - 2)
                def _():
                    fire(r0 + 2, 0)

                combine(r0 + 1, 1, goff)
                return carry2

            lax.fori_loop(0, L // 2, roi_pair, 0)
            return carry

        lax.fori_loop(0, g_per_w, group_body, 0)
        wait_out()

    return pl.kernel(
        body,
        out_type=(jax.ShapeDtypeStruct((n, CELLS, c), jnp.float32),
                  jax.ShapeDtypeStruct((CELLS, c), jnp.float32)),
        mesh=mesh,
        scratch_types=[
            pltpu.VMEM((4 * r_per_w,), jnp.float32),
            pltpu.VMEM((L,), jnp.float32),
            pltpu.VMEM((L * 2, IDXW), jnp.int32),
            pltpu.VMEM((PH, L), jnp.float32),
            pltpu.VMEM((PW, L), jnp.float32),
            pltpu.VMEM((IDXW, c), jnp.float32),
            pltpu.VMEM((IDXW, c), jnp.float32),
            pltpu.VMEM((IDXW, c), jnp.float32),
            pltpu.VMEM((IDXW, c), jnp.float32),
            pltpu.VMEM((CELLS, c), jnp.float32),
            pltpu.SemaphoreType.DMA,
            pltpu.SemaphoreType.DMA,
            pltpu.SemaphoreType.DMA,
        ],
        compiler_params=pltpu.CompilerParams(needs_layout_passes=False),
    )


def kernel(rois, feat_p2, feat_p3, feat_p4, feat_p5, img_metas):
    n = rois.shape[0]
    c = feat_p2.shape[-1]
    n_pad = ((n + NW * L - 1) // (NW * L)) * (NW * L)

    table = jnp.concatenate(
        [f.reshape(-1, c) for f in (feat_p2, feat_p3, feat_p4, feat_p5)],
        axis=0)
    rois_t = jnp.zeros((4, n_pad), jnp.float32).at[:, :n].set(
        rois.astype(jnp.float32).T).reshape(-1)
    pads = jnp.zeros((L,), jnp.float32).at[0].set(
        img_metas[0, 7]).at[1].set(img_metas[0, 8])

    out, _ = _build_sc_call(n, n_pad, c)(table, rois_t, pads)
    return out.reshape(n, PH, PW, c)


# speedup vs baseline: 1.5841x; 1.2374x over previous
import functools

import jax
import jax.numpy as jnp
import numpy as np
from jax import lax
from jax.experimental import pallas as pl
from jax.experimental.pallas import tpu as pltpu
from jax.experimental.pallas import tpu_sc as plsc

NC, NS, L = 2, 16, 16
NW = NC * NS
PH, PW = 7, 7
CELLS = PH * PW
IDXW = 104

HS = (256, 128, 64, 32)
BASES = (0, 65536, 81920, 86016)
ROWS = 87040
ODD_BASE = ROWS // 2 + 1

FRAC = tuple(float(np.float32(k) / np.float32(6.0)) for k in range(PH))


@functools.lru_cache(maxsize=None)
def _build_sc_call(n, n_pad, c):
    r_per_w = n_pad // NW
    g_per_w = r_per_w // L
    cj = c // L

    mesh = plsc.VectorSubcoreMesh(core_axis_name="c", subcore_axis_name="s",
                                  num_cores=NC, num_subcores=NS)

    def body(table, rois_t, pads, out, out_dummy, roi_v, pads_v, idx_buf,
             wy_buf, wx_buf, ab0, ab1, out_buf, sem0, sem1, osem):
        cid = lax.axis_index("c")
        sid = lax.axis_index("s")
        wid = sid * NC + cid
        base_roi = wid * r_per_w

        for i in range(4):
            pltpu.sync_copy(rois_t.at[pl.ds(i * n_pad + base_roi, r_per_w)],
                            roi_v.at[pl.ds(i * r_per_w, r_per_w)])
        pltpu.sync_copy(pads, pads_v)
        pads_vec = pads_v[...]
        area = pads_vec[0] * pads_vec[1]
        t3 = 224.0 * 224.0 * 0.125
        t4 = 224.0 * 224.0 * 0.5
        t5 = 224.0 * 224.0 * 2.0

        zero16 = jnp.zeros((L,), jnp.int32)
        for r in range(L):
            for off in (0, 16, 32, 48, 64, 80, 88):
                idx_buf[r, pl.ds(off, L)] = zero16

        lanes = lax.iota(jnp.int32, L)
        one = jnp.ones((L,), jnp.int32)
        sets = ((ab0, sem0), (ab1, sem1))

        def fire(rr, s):
            gbuf, sem = sets[s]
            pltpu.async_copy(table.at[idx_buf.at[rr]], gbuf, sem)

        def wait(s):
            gbuf, sem = sets[s]
            pltpu.make_async_copy(table.at[idx_buf.at[0]], gbuf, sem).wait()

        def wait_out():
            pltpu.make_async_copy(out_buf, out.at[0], osem).wait()

        def combine(rr, s, goff):
            gbuf, _ = sets[s]
            rs = jnp.full((L,), rr, jnp.int32)
            roi_g = base_roi + goff + rr

            @pl.when(goff + rr > 0)
            def _():
                wait_out()

            def cell_body(cc, carry3):
                pyv = cc // PW
                pxv = cc - pyv * PW
                wys = plsc.load_gather(
                    wy_buf, [jnp.full((L,), pyv, jnp.int32), rs])
                wxs = plsc.load_gather(
                    wx_buf, [jnp.full((L,), pxv, jnp.int32), rs])
                wd = wys * wxs
                wc = wys - wd
                wb = wxs - wd
                wa = (1.0 - wxs) - wc
                for j in range(cj):
                    sl = pl.ds(j * L, L)
                    sr = pl.ds(c + j * L, L)
                    av = gbuf[cc, sl]
                    bv = gbuf[cc, sr]
                    cv = gbuf[CELLS + cc, sl]
                    dv = gbuf[CELLS + cc, sr]
                    out_buf[cc, sl] = (av * wa + bv * wb + cv * wc + dv * wd)
                return carry3

            lax.fori_loop(0, CELLS, cell_body, 0)

            @pl.when(roi_g < n)
            def _():
                pltpu.async_copy(out_buf, out.at[roi_g], osem)

            @pl.when(roi_g >= n)
            def _():
                pltpu.async_copy(out_buf, out_dummy, osem)

        def group_body(g, carry):
            goff = g * L
            y1v = roi_v[pl.ds(goff, L)]
            x1v = roi_v[pl.ds(r_per_w + goff, L)]
            y2v = roi_v[pl.ds(2 * r_per_w + goff, L)]
            x2v = roi_v[pl.ds(3 * r_per_w + goff, L)]
            hv = y2v - y1v
            wv = x2v - x1v
            sv = hv * wv * area
            li = (jnp.where(sv >= t3, one, 0) + jnp.where(sv >= t4, one, 0)
                  + jnp.where(sv >= t5, one, 0))
            hm1f = jnp.where(li == 0, float(HS[0] - 1),
                             jnp.where(li == 1, float(HS[1] - 1),
                                       jnp.where(li == 2, float(HS[2] - 1),
                                                 float(HS[3] - 1))))
            hm1i = jnp.where(li == 0, HS[0] - 1,
                             jnp.where(li == 1, HS[1] - 1,
                                       jnp.where(li == 2, HS[2] - 1,
                                                 HS[3] - 1)))
            wint = hm1i + 1
            basev = jnp.where(li == 0, BASES[0],
                              jnp.where(li == 1, BASES[1],
                                        jnp.where(li == 2, BASES[2],
                                                  BASES[3])))
            hh = hv * hm1f
            ww = wv * hm1f
            y1s = y1v * hm1f
            x1s = x1v * hm1f

            ya = []
            yb = []
            x0 = []
            for k in range(PH):
                ys = y1s + FRAC[k] * hh
                y0i = ys.astype(jnp.int32)
                wy_buf[k] = ys - y0i.astype(jnp.float32)
                y1i = jnp.minimum(y0i + 1, hm1i)
                ya.append(basev + y0i * wint)
                yb.append(basev + y1i * wint)
                xs = x1s + FRAC[k] * ww
                x0i = xs.astype(jnp.int32)
                wx_buf[k] = xs - x0i.astype(jnp.float32)
                x0.append(x0i)

            for py in range(PH):
                for px in range(PW):
                    cc = py * PW + px
                    p_ab = ya[py] + x0[px]
                    p_cd = yb[py] + x0[px]
                    i_ab = (jnp.right_shift(p_ab, 1)
                            + jnp.bitwise_and(p_ab, one) * ODD_BASE)
                    i_cd = (jnp.right_shift(p_cd, 1)
                            + jnp.bitwise_and(p_cd, one) * ODD_BASE)
                    plsc.store_scatter(
                        idx_buf, [lanes, jnp.full((L,), cc, jnp.int32)], i_ab)
                    plsc.store_scatter(
                        idx_buf,
                        [lanes, jnp.full((L,), CELLS + cc, jnp.int32)], i_cd)

            fire(0, 0)

            def roi_pair(rh, carry2):
                r0 = rh * 2
                wait(0)
                fire(r0 + 1, 1)
                combine(r0, 0, goff)
                wait(1)

                @pl.when(r0 < L - 2)
                def _():
                    fire(r0 + 2, 0)

                combine(r0 + 1, 1, goff)
                return carry2

            lax.fori_loop(0, L // 2, roi_pair, 0)
            return carry

        lax.fori_loop(0, g_per_w, group_body, 0)
        wait_out()

    return pl.kernel(
        body,
        out_type=(jax.ShapeDtypeStruct((n, CELLS, c), jnp.float32),
                  jax.ShapeDtypeStruct((CELLS, c), jnp.float32)),
        mesh=mesh,
        scratch_types=[
            pltpu.VMEM((4 * r_per_w,), jnp.float32),
            pltpu.VMEM((L,), jnp.float32),
            pltpu.VMEM((L, IDXW), jnp.int32),
            pltpu.VMEM((PH, L), jnp.float32),
            pltpu.VMEM((PW, L), jnp.float32),
            pltpu.VMEM((IDXW, 2 * c), jnp.float32),
            pltpu.VMEM((IDXW, 2 * c), jnp.float32),
            pltpu.VMEM((CELLS, c), jnp.float32),
            pltpu.SemaphoreType.DMA,
            pltpu.SemaphoreType.DMA,
            pltpu.SemaphoreType.DMA,
        ],
        compiler_params=pltpu.CompilerParams(needs_layout_passes=False),
    )


def kernel(rois, feat_p2, feat_p3, feat_p4, feat_p5, img_metas):
    n = rois.shape[0]
    c = feat_p2.shape[-1]
    n_pad = ((n + NW * L - 1) // (NW * L)) * (NW * L)

    flat = jnp.concatenate(
        [f.reshape(-1, c) for f in (feat_p2, feat_p3, feat_p4, feat_p5)]
        + [jnp.zeros((2, c), feat_p2.dtype)], axis=0)
    table = jnp.concatenate(
        [flat.reshape(-1, 2 * c), flat[1:-1].reshape(-1, 2 * c)], axis=0)
    rois_t = jnp.zeros((4, n_pad), jnp.float32).at[:, :n].set(
        rois.astype(jnp.float32).T).reshape(-1)
    pads = jnp.zeros((L,), jnp.float32).at[0].set(
        img_metas[0, 7]).at[1].set(img_metas[0, 8])

    out, _ = _build_sc_call(n, n_pad, c)(table, rois_t, pads)
    return out.reshape(n, PH, PW, c)
